# baseline (speedup 1.0000x reference)
import functools

import jax
import jax.numpy as jnp
from jax import lax
from jax.experimental import pallas as pl
from jax.experimental.pallas import tpu as pltpu
from jax.experimental.pallas import tpu_sc as plsc

DIMS = (64, 32, 32, 16)
OFFS = (0, 64, 96, 128)
DTOT = 144
LCH = 128
NBUF = 5


def _sc_embed(xflat, table0, table1, table2, table3, nb, l):
    info = plsc.get_sparse_core_info()
    nc, ns = info.num_cores, info.num_subcores
    nw = nc * ns
    lk_w = nb * l // nw
    blks = lk_w // LCH
    rounds = blks // NBUF
    mesh = plsc.VectorSubcoreMesh(core_axis_name="c", subcore_axis_name="s")

    @functools.partial(
        pl.kernel,
        mesh=mesh,
        compiler_params=pltpu.CompilerParams(
            use_tc_tiling_on_sc=False, needs_layout_passes=False),
        out_type=jax.ShapeDtypeStruct((nb * l, DTOT), jnp.float32),
        scratch_types=(
            pltpu.VMEM((lk_w // 5, 4), jnp.int32),
            [pltpu.VMEM((blks, LCH), jnp.int32) for _ in range(4)],
            [[pltpu.VMEM((LCH, d), jnp.float32) for d in DIMS]
             for _ in range(NBUF)],
            [pltpu.SemaphoreType.DMA for _ in range(NBUF)],
            [pltpu.SemaphoreType.DMA for _ in range(NBUF)],
        ),
    )
    def k(x_hbm, t0, t1, t2, t3, out_hbm, xloc, idxs, embs, gsems, wsems):
        tabs = (t0, t1, t2, t3)
        wid = lax.axis_index("s") * nc + lax.axis_index("c")
        lk_base = wid * lk_w

        seg = blks // 5
        for h in range(5):
            pltpu.sync_copy(
                x_hbm.at[pl.ds(lk_base + h * (lk_w // 5), lk_w // 5)], xloc)

            def tr(c, carry, h=h):
                for o in range(0, LCH, 16):
                    rows = c * LCH + o + lax.iota(jnp.int32, 16)
                    for f in range(4):
                        v = plsc.load_gather(
                            xloc, [rows, jnp.full((16,), f, jnp.int32)])
                        idxs[f][h * seg + c, pl.ds(o, 16)] = v
                return carry

            lax.fori_loop(0, seg, tr, 0)

        def wr(b, c, f):
            return pltpu.make_async_copy(
                embs[b][f],
                out_hbm.at[pl.ds(lk_base + c * LCH, LCH),
                           pl.ds(OFFS[f], DIMS[f])],
                wsems[b])

        def outer(r, carry):
            c0 = r * NBUF
            for b in range(NBUF):
                c = c0 + b
                @pl.when(r > 0)
                def _(b=b, c=c):
                    for f in range(4):
                        wr(b, c, f).wait()
                for f in range(4):
                    pltpu.async_copy(
                        tabs[f].at[idxs[f].at[c]], embs[b][f], gsems[b])
            for b in range(NBUF):
                c = c0 + b
                for f in range(4):
                    pltpu.make_async_copy(
                        tabs[f].at[idxs[f].at[c]], embs[b][f], gsems[b]).wait()
                for f in range(4):
                    hit = jnp.zeros((16,), jnp.bool_)
                    for o in range(0, LCH, 16):
                        hit = jnp.logical_or(
                            hit, idxs[f][c, pl.ds(o, 16)] == 0)

                    @pl.when(jnp.any(hit))
                    def _(b=b, c=c, f=f):
                        def zero_group(g, carry2):
                            o = g * 16
                            iv = idxs[f][c, pl.ds(o, 16)]
                            rows = o + lax.iota(jnp.int32, 16)
                            z = jnp.zeros((16,), jnp.float32)
                            for col in range(DIMS[f]):
                                plsc.store_scatter(
                                    embs[b][f],
                                    [rows, jnp.full((16,), col, jnp.int32)],
                                    z, mask=iv == 0)
                            return carry2
                        lax.fori_loop(0, LCH // 16, zero_group, 0)
                for f in range(4):
                    wr(b, c, f).start()
            return carry

        lax.fori_loop(0, rounds, outer, 0)

        for b in range(NBUF):
            for f in range(4):
                wr(b, b, f).wait()

    return k(xflat, table0, table1, table2, table3).reshape(nb, l, DTOT)


def kernel(x, table0, table1, table2, table3):
    nb, l, f = x.shape
    info = plsc.get_sparse_core_info()
    nw = info.num_cores * info.num_subcores
    lk_w = nb * l // nw
    return _sc_embed(x.reshape(nb * l, f), table0, table1, table2, table3,
                     nb, l)


# speedup vs baseline: 1.3178x; 1.3178x over previous
import functools

import jax
import jax.numpy as jnp
from jax import lax
from jax.experimental import pallas as pl
from jax.experimental.pallas import tpu as pltpu
from jax.experimental.pallas import tpu_sc as plsc

DIMS = (64, 32, 32, 16)
OFFS = (0, 64, 96, 128)
DTOT = 144
LCH = 128
NBUF = 5


def _sc_embed(xt4, table0, table1, table2, table3, nb, l):
    info = plsc.get_sparse_core_info()
    nc, ns = info.num_cores, info.num_subcores
    nw = nc * ns
    lk_w = nb * l // nw
    blks = lk_w // LCH
    rounds = blks // NBUF
    mesh = plsc.VectorSubcoreMesh(core_axis_name="c", subcore_axis_name="s")

    @functools.partial(
        pl.kernel,
        mesh=mesh,
        compiler_params=pltpu.CompilerParams(
            use_tc_tiling_on_sc=False, needs_layout_passes=False),
        out_type=jax.ShapeDtypeStruct((nb * l, DTOT), jnp.float32),
        scratch_types=(
            [pltpu.VMEM((blks, LCH), jnp.int32) for _ in range(4)],
            [[pltpu.VMEM((LCH, d), jnp.float32) for d in DIMS]
             for _ in range(NBUF)],
            [pltpu.SemaphoreType.DMA for _ in range(NBUF)],
            [pltpu.SemaphoreType.DMA for _ in range(NBUF)],
        ),
    )
    def k(xt_hbm, t0, t1, t2, t3, out_hbm, idxs, embs, gsems, wsems):
        tabs = (t0, t1, t2, t3)
        wid = lax.axis_index("s") * nc + lax.axis_index("c")
        lk_base = wid * lk_w

        for f in range(4):
            pltpu.sync_copy(xt_hbm.at[f, wid], idxs[f])

        def wr(b, c, f):
            return pltpu.make_async_copy(
                embs[b][f],
                out_hbm.at[pl.ds(lk_base + c * LCH, LCH),
                           pl.ds(OFFS[f], DIMS[f])],
                wsems[b])

        def outer(r, carry):
            c0 = r * NBUF
            for b in range(NBUF):
                c = c0 + b
                @pl.when(r > 0)
                def _(b=b, c=c):
                    for f in range(4):
                        wr(b, c, f).wait()
                for f in range(4):
                    pltpu.async_copy(
                        tabs[f].at[idxs[f].at[c]], embs[b][f], gsems[b])
            for b in range(NBUF):
                c = c0 + b
                for f in range(4):
                    pltpu.make_async_copy(
                        tabs[f].at[idxs[f].at[c]], embs[b][f], gsems[b]).wait()
                for f in range(4):
                    hit = jnp.zeros((16,), jnp.bool_)
                    for o in range(0, LCH, 16):
                        hit = jnp.logical_or(
                            hit, idxs[f][c, pl.ds(o, 16)] == 0)

                    @pl.when(jnp.any(hit))
                    def _(b=b, c=c, f=f):
                        def zero_group(g, carry2):
                            o = g * 16
                            iv = idxs[f][c, pl.ds(o, 16)]
                            rows = o + lax.iota(jnp.int32, 16)
                            z = jnp.zeros((16,), jnp.float32)
                            for col in range(DIMS[f]):
                                plsc.store_scatter(
                                    embs[b][f],
                                    [rows, jnp.full((16,), col, jnp.int32)],
                                    z, mask=iv == 0)
                            return carry2
                        lax.fori_loop(0, LCH // 16, zero_group, 0)
                for f in range(4):
                    wr(b, c, f).start()
            return carry

        lax.fori_loop(0, rounds, outer, 0)

        for b in range(NBUF):
            for f in range(4):
                wr(b, b, f).wait()

    return k(xt4, table0, table1, table2, table3).reshape(nb, l, DTOT)


def kernel(x, table0, table1, table2, table3):
    nb, l, f = x.shape
    info = plsc.get_sparse_core_info()
    nw = info.num_cores * info.num_subcores
    lk_w = nb * l // nw
    xt4 = (x.reshape(nb * l, f).T
           .reshape(f, nw, lk_w // LCH, LCH))
    return _sc_embed(xt4, table0, table1, table2, table3, nb, l)
